# baseline (speedup 1.0000x reference)
import functools

import jax
import jax.numpy as jnp
from jax import lax
from jax.experimental import pallas as pl
from jax.experimental.pallas import tpu as pltpu, tpu_sc as plsc

VOCAB = 100000
EMB_DIM = 300
D_PAD = 304
BATCH = 4096
MAX_WORDS = 30

_B = BATCH * MAX_WORDS

_NC, _NS = 2, 16
_NW = _NC * _NS
_BPW = _B // _NW
_CHUNK = 64
_NCHUNK = _BPW // _CHUNK
_NBUF = 2

_mesh = plsc.VectorSubcoreMesh(core_axis_name="c", subcore_axis_name="s")


def _compact(rows_v, pack_v):
    def row(t, carry):
        dst = t * EMB_DIM
        for j in range(0, EMB_DIM - 16, 16):
            pack_v[pl.ds(dst + j, 16)] = rows_v[t, pl.ds(j, 16)]
        pack_v[pl.ds(dst + EMB_DIM - 16, 16)] = rows_v[t, pl.ds(EMB_DIM - 16, 16)]
        return carry

    lax.fori_loop(0, _CHUNK, row, 0)


@functools.partial(
    pl.kernel,
    out_type=jax.ShapeDtypeStruct((_B * EMB_DIM,), jnp.float32),
    mesh=_mesh,
    scratch_types=[
        [pltpu.VMEM((_CHUNK,), jnp.int32) for _ in range(_NBUF)],
        [pltpu.VMEM((_CHUNK, D_PAD), jnp.float32) for _ in range(_NBUF)],
        [pltpu.VMEM((_CHUNK * EMB_DIM,), jnp.float32) for _ in range(_NBUF)],
        [pltpu.SemaphoreType.DMA for _ in range(_NBUF)],
        [pltpu.SemaphoreType.DMA for _ in range(_NBUF)],
    ],
    compiler_params=pltpu.CompilerParams(use_tc_tiling_on_sc=False),
)
def _gather_kernel(idx_hbm, table_hbm, out_hbm, idx_v, rows_v, pack_v, gsem, wsem):
    wid = lax.axis_index("s") * _NC + lax.axis_index("c")
    base = wid * _BPW

    def start_gather(c, b):
        pltpu.sync_copy(idx_hbm.at[pl.ds(base + c * _CHUNK, _CHUNK)], idx_v[b])
        pltpu.async_copy(table_hbm.at[idx_v[b]], rows_v[b], gsem[b])

    def gather_wait(b):
        pltpu.make_async_copy(table_hbm.at[idx_v[b]], rows_v[b], gsem[b]).wait()

    def start_write(c, b):
        pltpu.async_copy(
            pack_v[b],
            out_hbm.at[pl.ds((base + c * _CHUNK) * EMB_DIM, _CHUNK * EMB_DIM)],
            wsem[b],
        )

    def write_wait(c, b):
        pltpu.make_async_copy(
            pack_v[b],
            out_hbm.at[pl.ds((base + c * _CHUNK) * EMB_DIM, _CHUNK * EMB_DIM)],
            wsem[b],
        ).wait()

    for b in range(_NBUF):
        start_gather(b, b)

    def step(g, carry):
        for b in range(_NBUF):
            c = g * _NBUF + b
            gather_wait(b)
            lax.cond(
                c >= _NBUF,
                lambda: write_wait(c - _NBUF, b),
                lambda: None,
            )
            _compact(rows_v[b], pack_v[b])
            start_write(c, b)
            lax.cond(
                c + _NBUF < _NCHUNK,
                lambda: start_gather(c + _NBUF, b),
                lambda: None,
            )
        return carry

    lax.fori_loop(0, _NCHUNK // _NBUF, step, 0)

    for b in range(_NBUF):
        write_wait(_NCHUNK - _NBUF + b, b)


def kernel(text, table):
    flat = text.reshape(-1).astype(jnp.int32)
    table_p = jnp.pad(table, ((0, 0), (0, D_PAD - EMB_DIM)))
    out = _gather_kernel(flat, table_p)
    return out.reshape(text.shape[:-1] + (MAX_WORDS, EMB_DIM))


# speedup vs baseline: 1.4070x; 1.4070x over previous
import functools

import jax
import jax.numpy as jnp
from jax import lax
from jax.experimental import pallas as pl
from jax.experimental.pallas import tpu as pltpu, tpu_sc as plsc

VOCAB = 100000
EMB_DIM = 300
D_PAD = 384
BATCH = 4096
MAX_WORDS = 30

_NC, _NS = 2, 16
_NW = _NC * _NS
_BATCH_PW = BATCH // _NW
_PAIR_B = 4
_PAIR_T = _PAIR_B * MAX_WORDS
_NPAIR = _BATCH_PW // _PAIR_B
_GROWS = 64

_OFFS = list(range(0, 256, 16)) + [256, 272, 284]

_mesh = plsc.VectorSubcoreMesh(core_axis_name="c", subcore_axis_name="s")


@functools.partial(
    pl.kernel,
    out_type=jax.ShapeDtypeStruct((BATCH, MAX_WORDS, EMB_DIM), jnp.float32),
    mesh=_mesh,
    scratch_types=[
        pltpu.VMEM((2 * _PAIR_T,), jnp.int32),
        pltpu.VMEM((_GROWS, D_PAD), jnp.float32),
        pltpu.VMEM((_GROWS, D_PAD), jnp.float32),
        pltpu.VMEM((2, MAX_WORDS, EMB_DIM), jnp.float32),
        pltpu.VMEM((2, MAX_WORDS, EMB_DIM), jnp.float32),
        pltpu.SemaphoreType.DMA,
        pltpu.SemaphoreType.DMA,
        pltpu.SemaphoreType.DMA,
        pltpu.SemaphoreType.DMA,
    ],
    compiler_params=pltpu.CompilerParams(use_tc_tiling_on_sc=True),
)
def _gather_kernel(idx_hbm, table_hbm, out_hbm,
                   idx_v, rows_a, rows_b, slab_a, slab_b,
                   gsem_a, gsem_b, wsem_a, wsem_b):
    wid = lax.axis_index("s") * _NC + lax.axis_index("c")
    tok0 = wid * _BATCH_PW * MAX_WORDS
    b0 = wid * _BATCH_PW

    def load_idx(p):
        pltpu.sync_copy(
            idx_hbm.at[pl.ds(tok0 + p * _PAIR_T, _PAIR_T)],
            idx_v.at[pl.ds((p % 2) * _PAIR_T, _PAIR_T)],
        )

    def start_gather(p, half, rows, sem):
        pltpu.async_copy(
            table_hbm.at[idx_v.at[pl.ds((p % 2) * _PAIR_T + half * 56, _GROWS)]],
            rows, sem,
        )

    def wait_gather(p, half, rows, sem):
        pltpu.make_async_copy(
            table_hbm.at[idx_v.at[pl.ds((p % 2) * _PAIR_T + half * 56, _GROWS)]],
            rows, sem,
        ).wait()

    def repack(rows, rofs, slab):
        def tok(t, carry):
            b = t // MAX_WORDS
            w = t % MAX_WORDS
            for o in _OFFS:
                slab[b, w, pl.ds(o, 16)] = rows[t + rofs, pl.ds(o, 16)]
            return carry

        lax.fori_loop(0, 2 * MAX_WORDS, tok, 0)

    def start_write(p, half, slab, sem):
        pltpu.async_copy(
            slab, out_hbm.at[pl.ds(b0 + p * _PAIR_B + half * 2, 2)], sem
        )

    def wait_write(p, half, slab, sem):
        pltpu.make_async_copy(
            slab, out_hbm.at[pl.ds(b0 + p * _PAIR_B + half * 2, 2)], sem
        ).wait()

    load_idx(0)
    start_gather(0, 0, rows_a, gsem_a)
    start_gather(0, 1, rows_b, gsem_b)

    def pair(p, carry):
        wait_gather(p, 0, rows_a, gsem_a)
        lax.cond(p > 0, lambda: wait_write(p - 1, 0, slab_a, wsem_a), lambda: None)
        repack(rows_a, 0, slab_a)
        start_write(p, 0, slab_a, wsem_a)

        lax.cond(p + 1 < _NPAIR, lambda: load_idx(p + 1), lambda: None)
        lax.cond(
            p + 1 < _NPAIR,
            lambda: start_gather(p + 1, 0, rows_a, gsem_a),
            lambda: None,
        )

        wait_gather(p, 1, rows_b, gsem_b)
        lax.cond(p > 0, lambda: wait_write(p - 1, 1, slab_b, wsem_b), lambda: None)
        repack(rows_b, 4, slab_b)
        start_write(p, 1, slab_b, wsem_b)
        lax.cond(
            p + 1 < _NPAIR,
            lambda: start_gather(p + 1, 1, rows_b, gsem_b),
            lambda: None,
        )
        return carry

    lax.fori_loop(0, _NPAIR, pair, 0)

    wait_write(_NPAIR - 1, 0, slab_a, wsem_a)
    wait_write(_NPAIR - 1, 1, slab_b, wsem_b)


def kernel(text, table):
    flat = text.reshape(-1).astype(jnp.int32)
    table_p = jnp.pad(table, ((0, 0), (0, D_PAD - EMB_DIM)))
    out = _gather_kernel(flat, table_p)
    return out.reshape(text.shape[:-1] + (MAX_WORDS, EMB_DIM))
